# baseline (speedup 1.0000x reference)
import jax
import jax.numpy as jnp
from jax import lax
from jax.experimental import pallas as pl
from jax.experimental.pallas import tpu as pltpu

_N = 50
_F = 10
_DROP = _F + 1
_D = 65536
_H = _D // 2
_BD = 4096
_NB = _H // _BD


def _krum_kernel(y_ref, out_ref, yt_s, g_s):
    j = pl.program_id(0)
    y_blk = y_ref[...]
    yt_s[:, pl.ds(j * _BD, _BD)] = jnp.transpose(y_blk)
    part = lax.dot_general(y_blk, y_blk, (((0,), (0,)), ((), ())),
                           preferred_element_type=jnp.float32)

    @pl.when(j == 0)
    def _():
        g_s[...] = part

    @pl.when(j > 0)
    def _():
        g_s[...] = g_s[...] + part

    @pl.when(j == _NB - 1)
    def _():
        gp = g_s[...]
        g = gp[0:_N, 0:_N] + gp[_N:2 * _N, _N:2 * _N]
        rows = lax.broadcasted_iota(jnp.int32, (_N, _N), 0)
        cols = lax.broadcasted_iota(jnp.int32, (_N, _N), 1)
        eye = rows == cols
        diag = jnp.where(eye, g, 0.0)
        sq_col = jnp.sum(diag, axis=1, keepdims=True)
        sq_row = jnp.sum(diag, axis=0, keepdims=True)
        d2 = jnp.maximum(sq_col + sq_row - 2.0 * g, 0.0)
        dist = jnp.sqrt(d2)

        total = jnp.sum(dist, axis=1, keepdims=True)
        rem = dist
        for _ in range(_DROP):
            m = jnp.max(rem, axis=1, keepdims=True)
            hit = rem == m
            first = jnp.min(jnp.where(hit, cols, _N), axis=1, keepdims=True)
            rem = jnp.where(cols == first, -1.0, rem)
            total = total - m
        scores = total

        mn = jnp.min(scores)
        ridx = lax.broadcasted_iota(jnp.int32, (_N, 1), 0)
        i_star = jnp.min(jnp.where(scores == mn, ridx, _N))

        out_ref[0:1, :] = yt_s[pl.ds(i_star, 1), :]
        out_ref[1:2, :] = yt_s[pl.ds(i_star + _N, 1), :]


def kernel(input):
    y = input.reshape(_H, 2 * _N)
    out = pl.pallas_call(
        _krum_kernel,
        grid=(_NB,),
        in_specs=[pl.BlockSpec((_BD, 2 * _N), lambda j: (j, 0))],
        out_specs=pl.BlockSpec((2, _H), lambda j: (0, 0)),
        out_shape=jax.ShapeDtypeStruct((2, _H), jnp.float32),
        scratch_shapes=[
            pltpu.VMEM((2 * _N, _H), jnp.float32),
            pltpu.VMEM((2 * _N, 2 * _N), jnp.float32),
        ],
    )(y)
    return out.T.reshape(1, _D, 1)


# speedup vs baseline: 2.1419x; 2.1419x over previous
import jax
import jax.numpy as jnp
from jax import lax
from jax.experimental import pallas as pl
from jax.experimental.pallas import tpu as pltpu

_N = 50
_F = 10
_DROP = _F + 1
_D = 65536
_BD = 16384
_NB = _D // _BD


def _krum_kernel(x_ref, out_ref, xt_s, g_s):
    j = pl.program_id(0)
    x_blk = x_ref[...]
    xt_s[:, pl.ds(j * _BD, _BD)] = jnp.transpose(x_blk)
    part = lax.dot_general(x_blk, x_blk, (((0,), (0,)), ((), ())),
                           preferred_element_type=jnp.float32)

    @pl.when(j == 0)
    def _():
        g_s[...] = part

    @pl.when(j > 0)
    def _():
        g_s[...] = g_s[...] + part

    @pl.when(j == _NB - 1)
    def _():
        g = g_s[...]
        rows = lax.broadcasted_iota(jnp.int32, (_N, _N), 0)
        cols = lax.broadcasted_iota(jnp.int32, (_N, _N), 1)
        eye = rows == cols
        diag = jnp.where(eye, g, 0.0)
        sq_col = jnp.sum(diag, axis=1, keepdims=True)
        sq_row = jnp.sum(diag, axis=0, keepdims=True)
        d2 = jnp.maximum(sq_col + sq_row - 2.0 * g, 0.0)
        dist = jnp.sqrt(d2)

        total = jnp.sum(dist, axis=1, keepdims=True)
        rem = dist
        for _ in range(_DROP):
            m = jnp.max(rem, axis=1, keepdims=True)
            hit = rem == m
            first = jnp.min(jnp.where(hit, cols, _N), axis=1, keepdims=True)
            rem = jnp.where(cols == first, -1.0, rem)
            total = total - m
        scores = total

        mn = jnp.min(scores)
        ridx = lax.broadcasted_iota(jnp.int32, (_N, 1), 0)
        i_star = jnp.min(jnp.where(scores == mn, ridx, _N))
        out_ref[...] = xt_s[pl.ds(i_star, 1), :]


def kernel(input):
    x = input.reshape(_D, _N)
    out = pl.pallas_call(
        _krum_kernel,
        grid=(_NB,),
        in_specs=[pl.BlockSpec((_BD, _N), lambda j: (j, 0))],
        out_specs=pl.BlockSpec((1, _D), lambda j: (0, 0)),
        out_shape=jax.ShapeDtypeStruct((1, _D), jnp.float32),
        scratch_shapes=[
            pltpu.VMEM((_N, _D), jnp.float32),
            pltpu.VMEM((_N, _N), jnp.float32),
        ],
    )(x)
    return out.reshape(1, _D, 1)
